# Initial kernel scaffold; baseline (speedup 1.0000x reference)
import jax
import jax.numpy as jnp
from jax.experimental import pallas as pl


def kernel(x, edge_index, W0, b0, W1, b1, W2, b2):
    raise NotImplementedError("write your pallas kernel here")


# speedup vs baseline: 2.2023x; 2.2023x over previous
import jax
import jax.numpy as jnp
from jax import lax
from jax.experimental import pallas as pl
from jax.experimental.pallas import tpu as pltpu
from jax.experimental.pallas import tpu_sc as plsc

N_NODES = 10000
N_PAD = 10240
N_EDGES = 160000
RPC = 160
N_CHUNKS = N_PAD // RPC
ACC_ROWS = RPC + 8
DUMMY_ROW = RPC
KB = 64
NB_TOT = N_EDGES // KB + 2 * N_CHUNKS


def _make_agg_body(d):
    def _agg_body(table, srcb, ldstb, meta, out, src_v, ldst_v, rows_v, meta_v,
                  acc, sem):
        cid = lax.axis_index("c")
        sid = lax.axis_index("s")
        w = cid * 16 + sid
        ldr = ldst_v.at[:]
        for p in range(2):
            chunk = p * 32 + w
            base = chunk * RPC
            pltpu.sync_copy(meta.at[chunk], meta_v)
            mv = meta_v[...]
            bs, be = mv[0], mv[1]
            pltpu.sync_copy(table.at[pl.ds(base, RPC)], acc.at[pl.ds(0, RPC)])

            def body(i, carry):
                b = bs + i
                pltpu.sync_copy(srcb.at[b], src_v)
                pltpu.sync_copy(ldstb.at[b], ldst_v)
                pltpu.async_copy(table.at[src_v], rows_v, sem).wait()

                def grp(g, carry2):
                    ld = ldr[pl.ds(g * 16, 16)]
                    for k in range(16):
                        r = ld[k]
                        e = g * 16 + k
                        for j in range(d // 16):
                            sl = pl.ds(j * 16, 16)
                            acc[r, sl] += rows_v[e, sl]
                    return carry2

                lax.fori_loop(0, KB // 16, grp, 0)
                return carry

            lax.fori_loop(0, be - bs, body, 0)
            pltpu.sync_copy(acc.at[pl.ds(0, RPC)], out.at[pl.ds(base, RPC)])

    return _agg_body


def _make_agg(d):
    return pl.kernel(
        _make_agg_body(d),
        out_type=jax.ShapeDtypeStruct((N_PAD, d), jnp.float32),
        mesh=plsc.VectorSubcoreMesh(core_axis_name="c", subcore_axis_name="s"),
        scratch_types=[
            pltpu.VMEM((KB,), jnp.int32),
            pltpu.VMEM((KB,), jnp.int32),
            pltpu.VMEM((KB, d), jnp.float32),
            pltpu.VMEM((16,), jnp.int32),
            pltpu.VMEM((ACC_ROWS, d), jnp.float32),
            pltpu.SemaphoreType.DMA,
        ],
    )


def _degcnt_body(ldstb, meta, out, ldst_v, meta_v, acc):
    cid = lax.axis_index("c")
    sid = lax.axis_index("s")
    w = cid * 16 + sid
    ldr = ldst_v.at[:]
    ones = jnp.ones((16,), jnp.float32)
    sl0 = pl.ds(0, 16)
    for p in range(2):
        chunk = p * 32 + w
        base = chunk * RPC
        pltpu.sync_copy(meta.at[chunk], meta_v)
        mv = meta_v[...]
        bs, be = mv[0], mv[1]

        def zero(i, carry):
            acc[i, sl0] = jnp.zeros((16,), jnp.float32)
            return carry

        lax.fori_loop(0, ACC_ROWS, zero, 0)

        def body(i, carry):
            b = bs + i
            pltpu.sync_copy(ldstb.at[b], ldst_v)

            def grp(g, carry2):
                ld = ldr[pl.ds(g * 16, 16)]
                for k in range(16):
                    r = ld[k]
                    acc[r, sl0] += ones
                return carry2

            lax.fori_loop(0, KB // 16, grp, 0)
            return carry

        lax.fori_loop(0, be - bs, body, 0)
        pltpu.sync_copy(acc.at[pl.ds(0, RPC)], out.at[pl.ds(base, RPC)])


def _make_degcnt():
    return pl.kernel(
        _degcnt_body,
        out_type=jax.ShapeDtypeStruct((N_PAD, 16), jnp.float32),
        mesh=plsc.VectorSubcoreMesh(core_axis_name="c", subcore_axis_name="s"),
        scratch_types=[
            pltpu.VMEM((KB,), jnp.int32),
            pltpu.VMEM((16,), jnp.int32),
            pltpu.VMEM((ACC_ROWS, 16), jnp.float32),
        ],
    )


def _scale_body(cnt_ref, x_ref, o_ref):
    dis = lax.rsqrt(cnt_ref[...] + 1.0)
    o_ref[...] = x_ref[...] * dis


def _scale(cnt, x):
    m, d = x.shape
    bm = 2048
    return pl.pallas_call(
        _scale_body,
        grid=(m // bm,),
        in_specs=[pl.BlockSpec((bm, 1), lambda i: (i, 0)),
                  pl.BlockSpec((bm, d), lambda i: (i, 0))],
        out_specs=pl.BlockSpec((bm, d), lambda i: (i, 0)),
        out_shape=jax.ShapeDtypeStruct((m, d), jnp.float32),
    )(cnt, x)


def _mm01_body(cnt_ref, a_ref, w0_ref, b0_ref, w1_ref, o_ref):
    dis = lax.rsqrt(cnt_ref[...] + 1.0)
    h = jnp.dot(a_ref[...] * dis, w0_ref[...],
                preferred_element_type=jnp.float32) + b0_ref[...]
    h = jnp.maximum(h, 0.0) * dis
    o_ref[...] = jnp.dot(h, w1_ref[...], preferred_element_type=jnp.float32)


def _mm01(cnt, a, w0, b0, w1):
    m, k = a.shape
    n0 = w0.shape[1]
    n1 = w1.shape[1]
    bm = 1024
    return pl.pallas_call(
        _mm01_body,
        grid=(m // bm,),
        in_specs=[pl.BlockSpec((bm, 1), lambda i: (i, 0)),
                  pl.BlockSpec((bm, k), lambda i: (i, 0)),
                  pl.BlockSpec((k, n0), lambda i: (0, 0)),
                  pl.BlockSpec((1, n0), lambda i: (0, 0)),
                  pl.BlockSpec((n0, n1), lambda i: (0, 0))],
        out_specs=pl.BlockSpec((bm, n1), lambda i: (i, 0)),
        out_shape=jax.ShapeDtypeStruct((m, n1), jnp.float32),
    )(cnt, a, w0, b0, w1)


def _mm2_body(cnt_ref, a_ref, b1_ref, w2_ref, o_ref):
    dis = lax.rsqrt(cnt_ref[...] + 1.0)
    h = jnp.maximum(a_ref[...] * dis + b1_ref[...], 0.0) * dis
    o_ref[...] = jnp.dot(h, w2_ref[...], preferred_element_type=jnp.float32)


def _mm2(cnt, a, b1, w2):
    m, k = a.shape
    n = w2.shape[1]
    bm = 1024
    return pl.pallas_call(
        _mm2_body,
        grid=(m // bm,),
        in_specs=[pl.BlockSpec((bm, 1), lambda i: (i, 0)),
                  pl.BlockSpec((bm, k), lambda i: (i, 0)),
                  pl.BlockSpec((1, k), lambda i: (0, 0)),
                  pl.BlockSpec((k, n), lambda i: (0, 0))],
        out_specs=pl.BlockSpec((bm, n), lambda i: (i, 0)),
        out_shape=jax.ShapeDtypeStruct((m, n), jnp.float32),
    )(cnt, a, b1, w2)


def _final_body(cnt_ref, a_ref, b2_ref, o_ref):
    dis = lax.rsqrt(cnt_ref[...] + 1.0)
    o_ref[...] = a_ref[...] * dis + b2_ref[...]


def _final(cnt, a, b2):
    m, d = a.shape
    bm = 2048
    return pl.pallas_call(
        _final_body,
        grid=(m // bm,),
        in_specs=[pl.BlockSpec((bm, 1), lambda i: (i, 0)),
                  pl.BlockSpec((bm, d), lambda i: (i, 0)),
                  pl.BlockSpec((1, d), lambda i: (0, 0))],
        out_specs=pl.BlockSpec((bm, d), lambda i: (i, 0)),
        out_shape=jax.ShapeDtypeStruct((m, d), jnp.float32),
    )(cnt, a, b2)


def kernel(x, edge_index, W0, b0, W1, b1, W2, b2):
    src = edge_index[0].astype(jnp.int32)
    dst = edge_index[1].astype(jnp.int32)

    chunk_id = dst // RPC
    onehot = (chunk_id[:, None] == jnp.arange(N_CHUNKS, dtype=jnp.int32)[None, :])
    rank = jnp.cumsum(onehot.astype(jnp.int32), axis=0)
    myrank = jnp.take_along_axis(rank, chunk_id[:, None], axis=1)[:, 0] - 1
    bucket_cnt = rank[-1]
    nb_c = (bucket_cnt + KB - 1) // KB
    nb_c = (nb_c + 1) // 2 * 2
    bstart = jnp.concatenate(
        [jnp.zeros((1,), jnp.int32), jnp.cumsum(nb_c).astype(jnp.int32)])
    meta = (jnp.zeros((N_CHUNKS, 16), jnp.int32)
            .at[:, 0].set(bstart[:N_CHUNKS])
            .at[:, 1].set(bstart[1:]))
    pos = bstart[chunk_id] * KB + myrank
    pad_fill = jnp.arange(NB_TOT * KB, dtype=jnp.int32) % N_NODES
    srcb = pad_fill.at[pos].set(src, unique_indices=True).reshape(NB_TOT, KB)
    ldstb = (jnp.full((NB_TOT * KB,), DUMMY_ROW, jnp.int32)
             .at[pos].set(dst - chunk_id * RPC, unique_indices=True)
             .reshape(NB_TOT, KB))

    x_pad = jnp.zeros((N_PAD, x.shape[1]), jnp.float32).at[:N_NODES].set(x)
    b0r = b0.reshape(1, -1)
    b1r = b1.reshape(1, -1)
    b2r = b2.reshape(1, -1)

    agg256 = _make_agg(256)
    agg512 = _make_agg(512)

    cnt = _make_degcnt()(ldstb, meta)[:, :1]
    xs = _scale(cnt, x_pad)
    agg0 = agg256(xs, srcb, ldstb, meta)
    u1 = _mm01(cnt, agg0, W0, b0r, W1)
    agg1 = agg512(u1, srcb, ldstb, meta)
    u2 = _mm2(cnt, agg1, b1r, W2)
    agg2 = agg256(u2, srcb, ldstb, meta)
    out = _final(cnt, agg2, b2r)
    return out[:N_NODES]
